# baseline (speedup 1.0000x reference)
import functools

import jax
import jax.numpy as jnp
from jax import lax
from jax.experimental import pallas as pl
from jax.experimental.pallas import tpu as pltpu
from jax.experimental.pallas import tpu_sc as plsc

_TN = 1024
_NC, _NS, _L = 2, 16, 16
_NW = _NC * _NS
_T_SC = 1024
_GRP = 2


def _tc_router_kernel(x_ref, w_ref, b_ref, vals_ref, idx_ref):
    x = x_ref[...]
    w = w_ref[...]
    b = b_ref[...]
    logits = jax.lax.dot_general(
        w, x, (((1,), (1,)), ((), ())),
        preferred_element_type=jnp.float32) + b
    m1 = jnp.max(logits, axis=0, keepdims=True)
    e16 = jax.lax.broadcasted_iota(jnp.int32, logits.shape, 0)
    i1 = jnp.min(jnp.where(logits == m1, e16, 16), axis=0, keepdims=True)
    masked = jnp.where(e16 == i1, -jnp.inf, logits)
    m2 = jnp.max(masked, axis=0, keepdims=True)
    i2 = jnp.min(jnp.where(masked == m2, e16, 16), axis=0, keepdims=True)
    s = jnp.sum(jnp.exp(logits - m1), axis=0, keepdims=True)
    vals_ref[...] = jnp.concatenate([1.0 / s, jnp.exp(m2 - m1) / s], axis=0)
    idx_ref[...] = jnp.concatenate([i1, i2], axis=0)


def _tc_route(x, W, b, n_tc):
    d_model = x.shape[1]
    n_experts = W.shape[0]
    return pl.pallas_call(
        _tc_router_kernel,
        grid=(n_tc // _TN,),
        in_specs=[
            pl.BlockSpec((_TN, d_model), lambda i: (i, 0)),
            pl.BlockSpec((n_experts, d_model), lambda i: (0, 0)),
            pl.BlockSpec((n_experts, 1), lambda i: (0, 0)),
        ],
        out_specs=[
            pl.BlockSpec((2, _TN), lambda i: (0, i)),
            pl.BlockSpec((2, _TN), lambda i: (0, i)),
        ],
        out_shape=[
            jax.ShapeDtypeStruct((2, n_tc), jnp.float32),
            jax.ShapeDtypeStruct((2, n_tc), jnp.int32),
        ],
    )(x, W, b.reshape(n_experts, 1))


def _round_bf16(v):
    u = plsc.bitcast(v, jnp.uint32)
    r = (u + jnp.uint32(0x7FFF) + ((u >> 16) & jnp.uint32(1))) \
        & jnp.uint32(0xFFFF0000)
    return plsc.bitcast(r, jnp.float32)


def _top2_write(ls, valv, idxv, sl):
    m1 = functools.reduce(jnp.maximum, ls)
    i1 = jnp.zeros((16,), jnp.int32)
    for e in range(15, -1, -1):
        i1 = jnp.where(ls[e] == m1, e, i1)
    neg_inf = jnp.full((16,), -jnp.inf, jnp.float32)
    l2 = [jnp.where(i1 == e, neg_inf, ls[e]) for e in range(16)]
    m2 = functools.reduce(jnp.maximum, l2)
    i2 = jnp.zeros((16,), jnp.int32)
    for e in range(15, -1, -1):
        i2 = jnp.where(l2[e] == m2, e, i2)
    s = None
    for e in range(16):
        t = jnp.exp(ls[e] - m1)
        s = t if s is None else s + t
    valv[0, sl] = 1.0 / s
    valv[1, sl] = jnp.exp(m2 - m1) / s
    idxv[0, sl] = i1
    idxv[1, sl] = i2


_UNROLL = 8


def _sc_router(x_hbm, w_hbm, b_hbm, vals_hbm, idx_hbm,
               xs, wvm, bv, valv, idxv, sem):
    n_tc = x_hbm.shape[0] - _T_SC
    d_model = x_hbm.shape[1]
    tpt = _T_SC // _NW
    wid = lax.axis_index("s") * _NC + lax.axis_index("c")
    base = n_tc + wid * tpt

    c1 = pltpu.async_copy(x_hbm.at[pl.ds(base, tpt)], xs, sem)
    c2 = pltpu.async_copy(w_hbm, wvm, sem)
    c3 = pltpu.async_copy(b_hbm, bv, sem)
    c1.wait(); c2.wait(); c3.wait()

    lane = lax.iota(jnp.int32, 16)
    bvec = bv[...]
    n_blk = tpt // (_GRP * _L)
    for blk in range(n_blk):
        tok0 = blk * _GRP * _L
        accs = []
        for grp in range(_GRP):
            accs.extend(jnp.full((16,), bvec[e]) for e in range(16))
        tokidx = [lane + (tok0 + grp * _L) for grp in range(_GRP)]

        def body(i, carry, tokidx=tokidx):
            dsplat = carry[0]
            accs = list(carry[1:])
            for _ in range(_UNROLL):
                wv = _round_bf16(plsc.load_gather(wvm, [lane, dsplat]))
                xcols = [
                    _round_bf16(plsc.load_gather(xs, [tokidx[g], dsplat]))
                    for g in range(_GRP)
                ]
                for e in range(16):
                    ws = jnp.full((16,), wv[e])
                    for g in range(_GRP):
                        accs[g * 16 + e] = accs[g * 16 + e] + xcols[g] * ws
                dsplat = dsplat + 1
            return (dsplat, *accs)

        carry = lax.fori_loop(
            0, d_model // _UNROLL, body,
            (jnp.zeros((16,), jnp.int32), *accs))
        accs = list(carry[1:])
        for grp in range(_GRP):
            ls = [accs[grp * 16 + e] for e in range(16)]
            _top2_write(ls, valv, idxv, pl.ds(tok0 + grp * _L, _L))

    tok = pl.ds(wid * tpt, tpt)
    pltpu.sync_copy(valv.at[0], vals_hbm.at[0, tok])
    pltpu.sync_copy(valv.at[1], vals_hbm.at[1, tok])
    pltpu.sync_copy(idxv.at[0], idx_hbm.at[0, tok])
    pltpu.sync_copy(idxv.at[1], idx_hbm.at[1, tok])


def _sc_route(x, W, b):
    d_model = x.shape[1]
    tpt = _T_SC // _NW
    mesh = plsc.VectorSubcoreMesh(
        core_axis_name="c", subcore_axis_name="s",
        num_cores=_NC, num_subcores=_NS)
    return pl.kernel(
        _sc_router,
        out_type=[
            jax.ShapeDtypeStruct((2, _T_SC), jnp.float32),
            jax.ShapeDtypeStruct((2, _T_SC), jnp.int32),
        ],
        mesh=mesh,
        compiler_params=pltpu.CompilerParams(
            use_tc_tiling_on_sc=False, needs_layout_passes=False),
        scratch_types=[
            pltpu.VMEM((tpt, d_model), jnp.float32),
            pltpu.VMEM((16, d_model), jnp.float32),
            pltpu.VMEM((16,), jnp.float32),
            pltpu.VMEM((2, tpt), jnp.float32),
            pltpu.VMEM((2, tpt), jnp.int32),
            pltpu.SemaphoreType.DMA,
        ],
    )(x, W, b)


def kernel(x, W, b):
    n_tokens = x.shape[0]
    n_tc = n_tokens - _T_SC
    vals_tc, idx_tc = _tc_route(x, W, b, n_tc)
    vals_sc, idx_sc = _sc_route(x, W, b)
    vals = jnp.concatenate([vals_tc, vals_sc], axis=1)
    idx = jnp.concatenate([idx_tc, idx_sc], axis=1)
    return (vals.T, idx.T)


# speedup vs baseline: 10.4038x; 10.4038x over previous
import jax
import jax.numpy as jnp
from jax.experimental import pallas as pl

_TN = 1024


def _router_kernel(x1_ref, x2_ref, w_ref, b_ref, vals_ref, idx_ref):
    w = w_ref[...]
    b = b_ref[...]
    kh = x1_ref.shape[1]
    nt = ((1,), (1,)), ((), ())
    logits = jax.lax.dot_general(
        w[:, :kh], x1_ref[...], nt, preferred_element_type=jnp.float32)
    logits += jax.lax.dot_general(
        w[:, kh:], x2_ref[...], nt, preferred_element_type=jnp.float32)
    logits += b
    m1 = jnp.max(logits, axis=0, keepdims=True)
    e16 = jax.lax.broadcasted_iota(jnp.int32, logits.shape, 0)
    i1 = jnp.min(jnp.where(logits == m1, e16, 16), axis=0, keepdims=True)
    masked = jnp.where(e16 == i1, -jnp.inf, logits)
    m2 = jnp.max(masked, axis=0, keepdims=True)
    i2 = jnp.min(jnp.where(masked == m2, e16, 16), axis=0, keepdims=True)
    s = jnp.sum(jnp.exp(logits - m1), axis=0, keepdims=True)
    v1 = 1.0 / s
    v2 = jnp.exp(m2 - m1) / s
    vals_ref[...] = jnp.concatenate([v1, v2], axis=0)
    idx_ref[...] = jnp.concatenate([i1, i2], axis=0)


def kernel(x, W, b):
    n_tokens, d_model = x.shape
    n_experts = W.shape[0]
    kh = d_model // 2
    grid = (n_tokens // _TN,)
    vals_t, idx_t = pl.pallas_call(
        _router_kernel,
        grid=grid,
        in_specs=[
            pl.BlockSpec((_TN, kh), lambda i: (i, 0)),
            pl.BlockSpec((_TN, kh), lambda i: (i, 1)),
            pl.BlockSpec((n_experts, d_model), lambda i: (0, 0)),
            pl.BlockSpec((n_experts, 1), lambda i: (0, 0)),
        ],
        out_specs=[
            pl.BlockSpec((2, _TN), lambda i: (0, i)),
            pl.BlockSpec((2, _TN), lambda i: (0, i)),
        ],
        out_shape=[
            jax.ShapeDtypeStruct((2, n_tokens), jnp.float32),
            jax.ShapeDtypeStruct((2, n_tokens), jnp.int32),
        ],
    )(x, x, W, b.reshape(n_experts, 1))
    return (vals_t.T, idx_t.T)
